# baseline (speedup 1.0000x reference)
import jax
import jax.numpy as jnp
from jax.experimental import pallas as pl

K = 4
B, M, NPTS = 8, 4096, 32
BM = B * M
G = 512


def _body(x_ref, y_ref, rcs_ref, vr_ref,
          a1_ref, b1_ref, g1_ref, be1_ref,
          w2_ref, b2_ref, g2_ref, be2_ref,
          out_ref):
    x = x_ref[...]
    y = y_ref[...]
    rcs = rcs_ref[...]
    vr = vr_ref[...]

    rng = jnp.hypot(x, y)
    az = jnp.arctan2(y, x)

    r_lo = jnp.min(rng, axis=0, keepdims=True)
    r_hi = jnp.max(rng, axis=0, keepdims=True)
    a_lo = jnp.min(az, axis=0, keepdims=True)
    a_hi = jnp.max(az, axis=0, keepdims=True)
    ur = (r_hi - r_lo) / K
    ua = (a_hi - a_lo) / K
    ur = jnp.where(ur == 0, 1.0, ur)
    ua = jnp.where(ua == 0, 1.0, ua)
    ridx = jnp.floor((rng - r_lo) / ur).astype(jnp.int32)
    aidx = jnp.floor((az - a_lo) / ua).astype(jnp.int32)
    ridx = jnp.clip(jnp.where(ridx == K, K - 1, ridx), 0, K - 1)
    aidx = jnp.clip(jnp.where(aidx == K, K - 1, aidx), 0, K - 1)
    flat = ridx * K + aidx

    cnt_rows, c1_rows, c2_rows = [], [], []
    for k in range(K * K):
        mask = flat == k
        cnt_rows.append(jnp.sum(mask.astype(jnp.float32), axis=0, keepdims=True))
        c1_rows.append(jnp.max(jnp.where(mask, rcs, 0.0), axis=0, keepdims=True))
        c2_rows.append(jnp.max(jnp.where(mask, vr, 0.0), axis=0, keepdims=True))
    ra = jnp.concatenate(cnt_rows + c1_rows + c2_rows, axis=0)

    h1 = jax.lax.dot_general(a1_ref[...], ra, (((1,), (0,)), ((), ())),
                             preferred_element_type=jnp.float32)
    h1 = h1 + b1_ref[...]

    hg = h1.reshape(8, 64, G)
    mean = jnp.mean(hg, axis=1, keepdims=True)
    d = hg - mean
    var = jnp.mean(d * d, axis=1, keepdims=True)
    hn = (d * jax.lax.rsqrt(var + 1e-5)).reshape(512, G)
    h = jnp.maximum(hn * g1_ref[...] + be1_ref[...], 0.0)

    o = jax.lax.dot_general(w2_ref[...], h, (((1,), (0,)), ((), ())),
                            preferred_element_type=jnp.float32)
    o = o + b2_ref[...]

    og = o.reshape(8, 8, G)
    mean2 = jnp.mean(og, axis=1, keepdims=True)
    d2 = og - mean2
    var2 = jnp.mean(d2 * d2, axis=1, keepdims=True)
    on = (d2 * jax.lax.rsqrt(var2 + 1e-5)).reshape(64, G)
    on = jnp.maximum(on * g2_ref[...] + be2_ref[...], 0.0)
    out_ref[...] = on.T


def _run(x, y, rcs, vr, a1, b1e, g1e, be1e, w2f, b2, g2, be2,
         interpret=False):
    grid = BM // G
    whole = lambda s: pl.BlockSpec(s, lambda i: (0, 0))
    return pl.pallas_call(
        _body,
        grid=(grid,),
        in_specs=[
            pl.BlockSpec((NPTS, G), lambda i: (0, i)),
            pl.BlockSpec((NPTS, G), lambda i: (0, i)),
            pl.BlockSpec((NPTS, G), lambda i: (0, i)),
            pl.BlockSpec((NPTS, G), lambda i: (0, i)),
            whole((512, 48)), whole((512, 1)), whole((512, 1)), whole((512, 1)),
            whole((64, 512)), whole((64, 1)), whole((64, 1)), whole((64, 1)),
        ],
        out_specs=pl.BlockSpec((G, 64), lambda i: (i, 0)),
        out_shape=jax.ShapeDtypeStruct((BM, 64), jnp.float32),
        interpret=interpret,
    )(x, y, rcs, vr, a1, b1e, g1e, be1e, w2f, b2, g2, be2)


def kernel(groups_xy, W1, b1, g1, be1, W2, b2, g2, be2):
    g = groups_xy.reshape(BM, NPTS, 6)
    x = g[:, :, 0].T
    y = g[:, :, 1].T
    rcs = g[:, :, 3].T
    vr = g[:, :, 5].T
    a1 = jnp.kron(W1.reshape(32, 3), jnp.eye(16, dtype=jnp.float32))
    rep = lambda v: jnp.repeat(v, 16).reshape(-1, 1)
    w2f = W2.reshape(64, 512)
    col = lambda v: v.reshape(-1, 1)
    out = _run(x, y, rcs, vr, a1, rep(b1), rep(g1), rep(be1),
               w2f, col(b2), col(g2), col(be2))
    return out.reshape(B, M, 64)
